# Initial kernel scaffold; baseline (speedup 1.0000x reference)
import jax
import jax.numpy as jnp
from jax.experimental import pallas as pl


def kernel(self_vecs, neigh_vecs, W):
    raise NotImplementedError("write your pallas kernel here")


# speedup vs baseline: 1.1524x; 1.1524x over previous
import functools

import jax
import jax.numpy as jnp
from jax.experimental import pallas as pl

N = 10000
DEG = 32
D = 128
DOUT = 128
BN = 400


def _body(self_ref, neigh_ref, w_ref, out_ref):
    s = jnp.sum(neigh_ref[...], axis=1) + self_ref[...]
    m = s * (1.0 / (DEG + 1))
    out_ref[...] = jnp.maximum(
        jnp.dot(m, w_ref[...], preferred_element_type=jnp.float32), 0.0
    )


@jax.jit
def kernel(self_vecs, neigh_vecs, W):
    grid = (N // BN,)
    return pl.pallas_call(
        _body,
        grid=grid,
        in_specs=[
            pl.BlockSpec((BN, D), lambda i: (i, 0)),
            pl.BlockSpec((BN, DEG, D), lambda i: (i, 0, 0)),
            pl.BlockSpec((D, DOUT), lambda i: (0, 0)),
        ],
        out_specs=pl.BlockSpec((BN, DOUT), lambda i: (i, 0)),
        out_shape=jax.ShapeDtypeStruct((N, DOUT), jnp.float32),
    )(self_vecs, neigh_vecs, W)
